# Initial kernel scaffold; baseline (speedup 1.0000x reference)
import jax
import jax.numpy as jnp
from jax.experimental import pallas as pl


def kernel(t, gamma):
    raise NotImplementedError("write your pallas kernel here")


# speedup vs baseline: 89.4508x; 89.4508x over previous
import functools

import jax
import jax.numpy as jnp
from jax import lax
from jax.experimental import pallas as pl
from jax.experimental.pallas import tpu as pltpu
from jax.experimental.pallas import tpu_sc as plsc

_NC = 2
_NS = 16
_NW = _NC * _NS
_L = 16
_TAB = 1024
_CHUNK = 12800


def _tc_tables(gpad2d):
    def body(g_ref, a_ref, s_ref):
        g = g_ref[...]
        a2 = jax.nn.sigmoid(-g)
        a_ref[...] = jnp.sqrt(a2)
        s_ref[...] = jnp.sqrt(1.0 - a2)

    return pl.pallas_call(
        body,
        out_shape=(jax.ShapeDtypeStruct((8, 128), jnp.float32),) * 2,
    )(gpad2d)


def _sc_gather(t_flat, gtab, atab, stab):
    n = t_flat.shape[0]
    per_w = n // _NW
    n_chunks = per_w // _CHUNK
    mesh = plsc.VectorSubcoreMesh(
        core_axis_name="c", subcore_axis_name="s",
        num_cores=_NC, num_subcores=_NS,
    )

    @functools.partial(
        pl.kernel,
        out_type=(jax.ShapeDtypeStruct((n,), jnp.float32),) * 3,
        mesh=mesh,
        compiler_params=pltpu.CompilerParams(needs_layout_passes=False),
        scratch_types=[
            pltpu.VMEM((_TAB,), jnp.float32),
            pltpu.VMEM((_TAB,), jnp.float32),
            pltpu.VMEM((_TAB,), jnp.float32),
            pltpu.VMEM((_CHUNK,), jnp.int32),
            pltpu.VMEM((_CHUNK,), jnp.float32),
            pltpu.VMEM((_CHUNK,), jnp.float32),
            pltpu.VMEM((_CHUNK,), jnp.float32),
        ],
    )
    def k(t_hbm, g_hbm, a_hbm, s_hbm, og_hbm, oa_hbm, os_hbm,
          gt_v, at_v, st_v, idx_v, og_v, oa_v, os_v):
        wid = lax.axis_index("s") * _NC + lax.axis_index("c")
        base = wid * per_w
        pltpu.sync_copy(g_hbm, gt_v)
        pltpu.sync_copy(a_hbm, at_v)
        pltpu.sync_copy(s_hbm, st_v)

        def chunk_body(ci, carry):
            off = base + ci * _CHUNK
            pltpu.sync_copy(t_hbm.at[pl.ds(off, _CHUNK)], idx_v)

            def vbody(j, c):
                sl = pl.ds(j * _L, _L)
                idx = idx_v[sl]
                og_v[sl] = plsc.load_gather(gt_v, [idx])
                oa_v[sl] = plsc.load_gather(at_v, [idx])
                os_v[sl] = plsc.load_gather(st_v, [idx])
                return c

            lax.fori_loop(0, _CHUNK // _L, vbody, 0, unroll=4)
            pltpu.sync_copy(og_v, og_hbm.at[pl.ds(off, _CHUNK)])
            pltpu.sync_copy(oa_v, oa_hbm.at[pl.ds(off, _CHUNK)])
            pltpu.sync_copy(os_v, os_hbm.at[pl.ds(off, _CHUNK)])
            return carry

        lax.fori_loop(0, n_chunks, chunk_body, 0)

    return k(t_flat, gtab, atab, stab)


def kernel(t, gamma):
    shape = t.shape
    t_flat = t.reshape(-1).astype(jnp.int32)
    gpad = jnp.zeros((_TAB,), jnp.float32).at[: gamma.shape[0]].set(gamma)
    atab, stab = _tc_tables(gpad.reshape(8, 128))
    og, oa, osig = _sc_gather(t_flat, gpad, atab.reshape(-1), stab.reshape(-1))
    return og.reshape(shape), oa.reshape(shape), osig.reshape(shape)


# speedup vs baseline: 151.8985x; 1.6981x over previous
import functools

import jax
import jax.numpy as jnp
from jax import lax
from jax.experimental import pallas as pl
from jax.experimental.pallas import tpu as pltpu
from jax.experimental.pallas import tpu_sc as plsc

_NC = 2
_NS = 16
_NW = _NC * _NS
_L = 16
_TAB = 1024
_CROWS = 32


def _tc_tables(gpad2d):
    def body(g_ref, a_ref, s_ref):
        g = g_ref[...]
        a2 = jax.nn.sigmoid(-g)
        a_ref[...] = jnp.sqrt(a2)
        s_ref[...] = jnp.sqrt(1.0 - a2)

    return pl.pallas_call(
        body,
        out_shape=(jax.ShapeDtypeStruct((8, 128), jnp.float32),) * 2,
    )(gpad2d)


def _sc_gather(t2d, gtab, atab, stab):
    rows, cols = t2d.shape
    rows_per_w = rows // _NW
    n_chunks = rows_per_w // _CROWS
    nwin = cols // _L + (1 if cols % _L else 0)
    mesh = plsc.VectorSubcoreMesh(
        core_axis_name="c", subcore_axis_name="s",
        num_cores=_NC, num_subcores=_NS,
    )

    @functools.partial(
        pl.kernel,
        out_type=(jax.ShapeDtypeStruct((rows, cols), jnp.float32),) * 3,
        mesh=mesh,
        compiler_params=pltpu.CompilerParams(needs_layout_passes=False),
        scratch_types=[
            pltpu.VMEM((_TAB,), jnp.float32),
            pltpu.VMEM((_TAB,), jnp.float32),
            pltpu.VMEM((_TAB,), jnp.float32),
            pltpu.VMEM((2, _CROWS, cols), jnp.int32),
            pltpu.VMEM((2, _CROWS, cols), jnp.float32),
            pltpu.VMEM((2, _CROWS, cols), jnp.float32),
            pltpu.VMEM((2, _CROWS, cols), jnp.float32),
            pltpu.SemaphoreType.DMA,
            pltpu.SemaphoreType.DMA,
            pltpu.SemaphoreType.DMA,
            pltpu.SemaphoreType.DMA,
        ],
    )
    def k(t_hbm, g_hbm, a_hbm, s_hbm, og_hbm, oa_hbm, os_hbm,
          gt_v, at_v, st_v, idx_v, og_v, oa_v, os_v,
          sin0, sin1, sout0, sout1):
        wid = lax.axis_index("s") * _NC + lax.axis_index("c")
        base = wid * rows_per_w
        pltpu.sync_copy(g_hbm, gt_v)
        pltpu.sync_copy(a_hbm, at_v)
        pltpu.sync_copy(s_hbm, st_v)
        sin = (sin0, sin1)
        sout = (sout0, sout1)

        def start_in(ci, b):
            return pltpu.async_copy(
                t_hbm.at[pl.ds(base + ci * _CROWS, _CROWS), :],
                idx_v.at[b], sin[b])

        def start_out(ci, b):
            r0 = base + ci * _CROWS
            d = []
            for hbm, v in ((og_hbm, og_v), (oa_hbm, oa_v), (os_hbm, os_v)):
                d.append(pltpu.async_copy(
                    v.at[b], hbm.at[pl.ds(r0, _CROWS), :], sout[b]))
            return d

        def compute(b):
            def row_body(r, carry):
                for c in range(nwin):
                    col = min(c * _L, cols - _L)
                    sl = pl.ds(col, _L)
                    idx = idx_v[b, r, sl]
                    og_v[b, r, sl] = plsc.load_gather(gt_v, [idx])
                    oa_v[b, r, sl] = plsc.load_gather(at_v, [idx])
                    os_v[b, r, sl] = plsc.load_gather(st_v, [idx])
                return carry
            lax.fori_loop(0, _CROWS, row_body, 0)

        in_d = {0: start_in(0, 0), 1: start_in(1, 1)}
        out_d = {}
        for ci in range(n_chunks):
            b = ci % 2
            in_d.pop(ci).wait()
            if ci >= 2:
                for d in out_d.pop(ci - 2):
                    d.wait()
            compute(b)
            out_d[ci] = start_out(ci, b)
            if ci + 2 < n_chunks:
                in_d[ci + 2] = start_in(ci + 2, b)
        for ci in (n_chunks - 2, n_chunks - 1):
            for d in out_d.pop(ci):
                d.wait()

    return k(t2d, gtab, atab, stab)


def kernel(t, gamma):
    t2d = t.astype(jnp.int32)
    gpad = jnp.zeros((_TAB,), jnp.float32).at[: gamma.shape[0]].set(gamma)
    atab, stab = _tc_tables(gpad.reshape(8, 128))
    og, oa, osig = _sc_gather(t2d, gpad, atab.reshape(-1), stab.reshape(-1))
    return og, oa, osig


# speedup vs baseline: 249.4807x; 1.6424x over previous
import functools

import jax
import jax.numpy as jnp
from jax import lax
from jax.experimental import pallas as pl
from jax.experimental.pallas import tpu as pltpu
from jax.experimental.pallas import tpu_sc as plsc

_NC = 2
_NS = 16
_NW = _NC * _NS
_L = 16
_CROWS = 32


def _sqrt16(x):
    i = plsc.bitcast(x, jnp.int32)
    y = plsc.bitcast(jnp.int32(0x5F3759DF) - (i >> 1), jnp.float32)
    for _ in range(3):
        y = y * (1.5 - 0.5 * x * y * y)
    return x * y


def _windows(n):
    cols = [c * _L for c in range(n // _L)]
    if n % _L:
        cols.append(n - _L)
    return cols


def _sc_all(t2d, gamma):
    rows, cols = t2d.shape
    tab_n = gamma.shape[0]
    rows_per_w = rows // _NW
    n_chunks = rows_per_w // _CROWS
    mesh = plsc.VectorSubcoreMesh(
        core_axis_name="c", subcore_axis_name="s",
        num_cores=_NC, num_subcores=_NS,
    )

    @functools.partial(
        pl.kernel,
        out_type=(jax.ShapeDtypeStruct((rows, cols), jnp.float32),) * 3,
        mesh=mesh,
        compiler_params=pltpu.CompilerParams(needs_layout_passes=False),
        scratch_types=[
            pltpu.VMEM((tab_n,), jnp.float32),
            pltpu.VMEM((tab_n,), jnp.float32),
            pltpu.VMEM((tab_n,), jnp.float32),
            pltpu.VMEM((2, _CROWS, cols), jnp.int32),
            pltpu.VMEM((2, _CROWS, cols), jnp.float32),
            pltpu.VMEM((2, _CROWS, cols), jnp.float32),
            pltpu.VMEM((2, _CROWS, cols), jnp.float32),
            pltpu.SemaphoreType.DMA,
            pltpu.SemaphoreType.DMA,
            pltpu.SemaphoreType.DMA,
            pltpu.SemaphoreType.DMA,
        ],
    )
    def k(t_hbm, g_hbm, og_hbm, oa_hbm, os_hbm,
          gt_v, at_v, st_v, idx_v, og_v, oa_v, os_v,
          sin0, sin1, sout0, sout1):
        wid = lax.axis_index("s") * _NC + lax.axis_index("c")
        base = wid * rows_per_w
        sin = (sin0, sin1)
        sout = (sout0, sout1)

        def start_in(ci, b):
            return pltpu.async_copy(
                t_hbm.at[pl.ds(base + ci * _CROWS, _CROWS), :],
                idx_v.at[b], sin[b])

        def wait_in(b):
            pltpu.make_async_copy(
                t_hbm.at[pl.ds(0, _CROWS), :], idx_v.at[b], sin[b]).wait()

        def start_out(ci, b):
            r0 = base + ci * _CROWS
            for hbm, v in ((og_hbm, og_v), (oa_hbm, oa_v), (os_hbm, os_v)):
                pltpu.async_copy(v.at[b], hbm.at[pl.ds(r0, _CROWS), :], sout[b])

        def wait_out(b):
            for hbm, v in ((og_hbm, og_v), (oa_hbm, oa_v), (os_hbm, os_v)):
                pltpu.make_async_copy(
                    v.at[b], hbm.at[pl.ds(0, _CROWS), :], sout[b]).wait()

        start_in(0, 0)
        start_in(1, 1)
        pltpu.sync_copy(g_hbm, gt_v)

        def tab_body(j, carry):
            sl = pl.ds(j * _L, _L)
            g = gt_v[sl]
            a2 = 1.0 / (1.0 + jnp.exp(g))
            at_v[sl] = _sqrt16(a2)
            st_v[sl] = _sqrt16(1.0 - a2)
            return carry
        lax.fori_loop(0, tab_n // _L, tab_body, 0)
        sl = pl.ds(tab_n - _L, _L)
        g = gt_v[sl]
        a2 = 1.0 / (1.0 + jnp.exp(g))
        at_v[sl] = _sqrt16(a2)
        st_v[sl] = _sqrt16(1.0 - a2)

        win_cols = _windows(cols)

        def compute(b):
            @plsc.parallel_loop(0, _CROWS, 1, unroll=2)
            def row_body(r):
                for col in win_cols:
                    sl = pl.ds(col, _L)
                    idx = idx_v[b, r, sl]
                    og_v[b, r, sl] = plsc.load_gather(gt_v, [idx])
                    oa_v[b, r, sl] = plsc.load_gather(at_v, [idx])
                    os_v[b, r, sl] = plsc.load_gather(st_v, [idx])

        def pair_body(g_i, carry):
            for b in range(2):
                ci = g_i * 2 + b
                wait_in(b)

                @pl.when(g_i >= 1)
                def _():
                    wait_out(b)

                compute(b)
                start_out(ci, b)

                @pl.when(ci + 2 < n_chunks)
                def _():
                    start_in(ci + 2, b)
            return carry

        lax.fori_loop(0, n_chunks // 2, pair_body, 0)
        wait_out(0)
        wait_out(1)

    return k(t2d, gamma)


def kernel(t, gamma):
    og, oa, osig = _sc_all(t.astype(jnp.int32), gamma.astype(jnp.float32))
    return og, oa, osig


# speedup vs baseline: 249.7105x; 1.0009x over previous
import functools

import jax
import jax.numpy as jnp
from jax import lax
from jax.experimental import pallas as pl
from jax.experimental.pallas import tpu as pltpu
from jax.experimental.pallas import tpu_sc as plsc

_NC = 2
_NS = 16
_NW = _NC * _NS
_L = 16
_CROWS = 32


def _sqrt16(x):
    i = plsc.bitcast(x, jnp.int32)
    y = plsc.bitcast(jnp.int32(0x5F3759DF) - (i >> 1), jnp.float32)
    for _ in range(3):
        y = y * (1.5 - 0.5 * x * y * y)
    return x * y


def _windows(n):
    cols = [c * _L for c in range(n // _L)]
    if n % _L:
        cols.append(n - _L)
    return cols


def _sc_all(t2d, gamma):
    rows, cols = t2d.shape
    tab_n = gamma.shape[0]
    rows_per_w = rows // _NW
    n_chunks = rows_per_w // _CROWS
    mesh = plsc.VectorSubcoreMesh(
        core_axis_name="c", subcore_axis_name="s",
        num_cores=_NC, num_subcores=_NS,
    )

    @functools.partial(
        pl.kernel,
        out_type=(jax.ShapeDtypeStruct((rows, cols), jnp.float32),) * 3,
        mesh=mesh,
        compiler_params=pltpu.CompilerParams(
            needs_layout_passes=False, use_tc_tiling_on_sc=True),
        scratch_types=[
            pltpu.VMEM((tab_n,), jnp.float32),
            pltpu.VMEM((tab_n,), jnp.float32),
            pltpu.VMEM((tab_n,), jnp.float32),
            pltpu.VMEM((2, _CROWS, cols), jnp.int32),
            pltpu.VMEM((2, _CROWS, cols), jnp.float32),
            pltpu.VMEM((2, _CROWS, cols), jnp.float32),
            pltpu.VMEM((2, _CROWS, cols), jnp.float32),
            pltpu.SemaphoreType.DMA,
            pltpu.SemaphoreType.DMA,
            pltpu.SemaphoreType.DMA,
            pltpu.SemaphoreType.DMA,
        ],
    )
    def k(t_hbm, g_hbm, og_hbm, oa_hbm, os_hbm,
          gt_v, at_v, st_v, idx_v, og_v, oa_v, os_v,
          sin0, sin1, sout0, sout1):
        wid = lax.axis_index("s") * _NC + lax.axis_index("c")
        base = wid * rows_per_w
        sin = (sin0, sin1)
        sout = (sout0, sout1)

        def start_in(ci, b):
            return pltpu.async_copy(
                t_hbm.at[pl.ds(base + ci * _CROWS, _CROWS), :],
                idx_v.at[b], sin[b])

        def wait_in(b):
            pltpu.make_async_copy(
                t_hbm.at[pl.ds(0, _CROWS), :], idx_v.at[b], sin[b]).wait()

        def start_out(ci, b):
            r0 = base + ci * _CROWS
            for hbm, v in ((og_hbm, og_v), (oa_hbm, oa_v), (os_hbm, os_v)):
                pltpu.async_copy(v.at[b], hbm.at[pl.ds(r0, _CROWS), :], sout[b])

        def wait_out(b):
            for hbm, v in ((og_hbm, og_v), (oa_hbm, oa_v), (os_hbm, os_v)):
                pltpu.make_async_copy(
                    v.at[b], hbm.at[pl.ds(0, _CROWS), :], sout[b]).wait()

        start_in(0, 0)
        start_in(1, 1)
        pltpu.sync_copy(g_hbm, gt_v)

        def tab_body(j, carry):
            sl = pl.ds(j * _L, _L)
            g = gt_v[sl]
            a2 = 1.0 / (1.0 + jnp.exp(g))
            at_v[sl] = _sqrt16(a2)
            st_v[sl] = _sqrt16(1.0 - a2)
            return carry
        lax.fori_loop(0, tab_n // _L, tab_body, 0)
        sl = pl.ds(tab_n - _L, _L)
        g = gt_v[sl]
        a2 = 1.0 / (1.0 + jnp.exp(g))
        at_v[sl] = _sqrt16(a2)
        st_v[sl] = _sqrt16(1.0 - a2)

        win_cols = _windows(cols)

        def compute(b):
            @plsc.parallel_loop(0, _CROWS, 1, unroll=2)
            def row_body(r):
                for col in win_cols:
                    sl = pl.ds(col, _L)
                    idx = idx_v[b, r, sl]
                    og_v[b, r, sl] = plsc.load_gather(gt_v, [idx])
                    oa_v[b, r, sl] = plsc.load_gather(at_v, [idx])
                    os_v[b, r, sl] = plsc.load_gather(st_v, [idx])

        def pair_body(g_i, carry):
            for b in range(2):
                ci = g_i * 2 + b
                wait_in(b)

                @pl.when(g_i >= 1)
                def _():
                    wait_out(b)

                compute(b)
                start_out(ci, b)

                @pl.when(ci + 2 < n_chunks)
                def _():
                    start_in(ci + 2, b)
            return carry

        lax.fori_loop(0, n_chunks // 2, pair_body, 0)
        wait_out(0)
        wait_out(1)

    return k(t2d, gamma)


def kernel(t, gamma):
    og, oa, osig = _sc_all(t.astype(jnp.int32), gamma.astype(jnp.float32))
    return og, oa, osig


# speedup vs baseline: 519.9160x; 2.0821x over previous
import functools

import jax
import jax.numpy as jnp
from jax import lax
from jax.experimental import pallas as pl
from jax.experimental.pallas import tpu as pltpu
from jax.experimental.pallas import tpu_sc as plsc

_NC = 2
_NS = 16
_NW = _NC * _NS
_L = 16
_CROWS = 40
_CCOLS = 256


def _sqrt16(x):
    i = plsc.bitcast(x, jnp.int32)
    y = plsc.bitcast(jnp.int32(0x5F3759DF) - (i >> 1), jnp.float32)
    for _ in range(3):
        y = y * (1.5 - 0.5 * x * y * y)
    return x * y


def _sc_all(t2d, gamma):
    rows, cols = t2d.shape
    tab_n = gamma.shape[0]
    cols_per_w = cols // _NW
    col_sub = cols_per_w // _CCOLS
    n_chunks = (rows // _CROWS) * col_sub
    nwin = _CCOLS // _L
    mesh = plsc.VectorSubcoreMesh(
        core_axis_name="c", subcore_axis_name="s",
        num_cores=_NC, num_subcores=_NS,
    )

    @functools.partial(
        pl.kernel,
        out_type=(jax.ShapeDtypeStruct((rows, cols), jnp.float32),) * 3,
        mesh=mesh,
        compiler_params=pltpu.CompilerParams(needs_layout_passes=False),
        scratch_types=[
            pltpu.VMEM((tab_n,), jnp.float32),
            pltpu.VMEM((tab_n,), jnp.float32),
            pltpu.VMEM((tab_n,), jnp.float32),
            pltpu.VMEM((2, _CROWS, _CCOLS), jnp.int32),
            pltpu.VMEM((2, _CROWS, _CCOLS), jnp.float32),
            pltpu.VMEM((2, _CROWS, _CCOLS), jnp.float32),
            pltpu.VMEM((2, _CROWS, _CCOLS), jnp.float32),
            pltpu.SemaphoreType.DMA,
            pltpu.SemaphoreType.DMA,
            pltpu.SemaphoreType.DMA,
            pltpu.SemaphoreType.DMA,
        ],
    )
    def k(t_hbm, g_hbm, og_hbm, oa_hbm, os_hbm,
          gt_v, at_v, st_v, idx_v, og_v, oa_v, os_v,
          sin0, sin1, sout0, sout1):
        wid = lax.axis_index("s") * _NC + lax.axis_index("c")
        col0 = wid * cols_per_w
        sin = (sin0, sin1)
        sout = (sout0, sout1)

        def start_in(ci, b):
            return pltpu.async_copy(
                t_hbm.at[pl.ds((ci // col_sub) * _CROWS, _CROWS),
                         pl.ds(col0 + (ci % col_sub) * _CCOLS, _CCOLS)],
                idx_v.at[b], sin[b])

        def wait_in(b):
            pltpu.make_async_copy(
                t_hbm.at[pl.ds(0, _CROWS), pl.ds(col0, _CCOLS)],
                idx_v.at[b], sin[b]).wait()

        def start_out(ci, b):
            r0 = (ci // col_sub) * _CROWS
            c0 = col0 + (ci % col_sub) * _CCOLS
            for hbm, v in ((og_hbm, og_v), (oa_hbm, oa_v), (os_hbm, os_v)):
                pltpu.async_copy(
                    v.at[b],
                    hbm.at[pl.ds(r0, _CROWS), pl.ds(c0, _CCOLS)],
                    sout[b])

        def wait_out(b):
            for hbm, v in ((og_hbm, og_v), (oa_hbm, oa_v), (os_hbm, os_v)):
                pltpu.make_async_copy(
                    v.at[b],
                    hbm.at[pl.ds(0, _CROWS), pl.ds(col0, _CCOLS)],
                    sout[b]).wait()

        start_in(0, 0)
        start_in(1, 1)
        pltpu.sync_copy(g_hbm, gt_v)

        def tab_body(j, carry):
            sl = pl.ds(j * _L, _L)
            g = gt_v[sl]
            a2 = 1.0 / (1.0 + jnp.exp(g))
            at_v[sl] = _sqrt16(a2)
            st_v[sl] = _sqrt16(1.0 - a2)
            return carry
        lax.fori_loop(0, tab_n // _L, tab_body, 0)
        sl = pl.ds(tab_n - _L, _L)
        g = gt_v[sl]
        a2 = 1.0 / (1.0 + jnp.exp(g))
        at_v[sl] = _sqrt16(a2)
        st_v[sl] = _sqrt16(1.0 - a2)

        def compute(b):
            @plsc.parallel_loop(0, _CROWS, 1, unroll=2)
            def row_body(r):
                for c in range(nwin):
                    sl = pl.ds(c * _L, _L)
                    idx = idx_v[b, r, sl]
                    og_v[b, r, sl] = plsc.load_gather(gt_v, [idx])
                    oa_v[b, r, sl] = plsc.load_gather(at_v, [idx])
                    os_v[b, r, sl] = plsc.load_gather(st_v, [idx])

        def pair_body(g_i, carry):
            for b in range(2):
                ci = g_i * 2 + b
                wait_in(b)

                @pl.when(g_i >= 1)
                def _():
                    wait_out(b)

                compute(b)
                start_out(ci, b)

                @pl.when(ci + 2 < n_chunks)
                def _():
                    start_in(ci + 2, b)
            return carry

        lax.fori_loop(0, n_chunks // 2, pair_body, 0)
        wait_out(0)
        wait_out(1)

    return k(t2d, gamma)


def kernel(t, gamma):
    og, oa, osig = _sc_all(t.astype(jnp.int32).T, gamma.astype(jnp.float32))
    return og.T, oa.T, osig.T
